# Initial kernel scaffold; baseline (speedup 1.0000x reference)
import jax
import jax.numpy as jnp
from jax.experimental import pallas as pl


def kernel(x, edge_index, edge_type, relation_weights, lin_W, lin_b, gamma, beta):
    raise NotImplementedError("write your pallas kernel here")


# speedup vs baseline: 6.6183x; 6.6183x over previous
import functools

import jax
import jax.numpy as jnp
from jax import lax
from jax.experimental import pallas as pl
from jax.experimental.pallas import tpu as pltpu
from jax.experimental.pallas import tpu_sc as plsc

NC = 2
NS = 16
LANES = 16
DEGW = 128
CHUNK = 80


def _make_tables(x, relation_weights, bn):
    n, d = x.shape
    r = relation_weights.shape[0]

    def body(x_ref, w_ref, o_ref):
        o_ref[0] = jnp.dot(x_ref[...], w_ref[0],
                           preferred_element_type=jnp.float32)

    return pl.pallas_call(
        body,
        grid=(r, n // bn),
        in_specs=[
            pl.BlockSpec((bn, d), lambda i, j: (j, 0)),
            pl.BlockSpec((1, d, d), lambda i, j: (i, 0, 0)),
        ],
        out_specs=pl.BlockSpec((1, bn, d), lambda i, j: (i, j, 0)),
        out_shape=jax.ShapeDtypeStruct((r, n, d), jnp.float32),
    )(x, relation_weights)


def _make_edge_sweep(np_, d2, ept):
    nchunks = ept // CHUNK
    rows_pt = np_ // NS
    zrounds = rows_pt // 16
    ztail = rows_pt - zrounds * 16

    mesh = plsc.VectorSubcoreMesh(core_axis_name="c", subcore_axis_name="s")

    @functools.partial(
        pl.kernel,
        out_type=[
            jax.ShapeDtypeStruct((NC, np_, d2), jnp.float32),
        ],
        mesh=mesh,
        scratch_types=[
            pltpu.VMEM((CHUNK,), jnp.int32),
            pltpu.VMEM((CHUNK,), jnp.int32),
            pltpu.VMEM((CHUNK, d2), jnp.float32),
            pltpu.VMEM_SHARED((np_, d2), jnp.float32),
            pltpu.SemaphoreType.DMA,
        ],
    )
    def sweep(hf, gsrc, dst, acc_out, gidx_v, dst_v, rows_v, acc_sh, sem):
        c = lax.axis_index("c")
        s = lax.axis_index("s")
        wid = s * NC + c

        zf = jnp.zeros((LANES,), jnp.float32)

        for j in range(16):
            for k in range(d2 // LANES):
                rows_v[j, pl.ds(k * LANES, LANES)] = zf

        row0 = s * rows_pt
        zrows = rows_v.at[pl.ds(0, 16)]

        def _zacc(i, carry):
            pltpu.sync_copy(zrows, acc_sh.at[pl.ds(row0 + i * 16, 16)])
            return carry
        lax.fori_loop(0, zrounds, _zacc, 0)
        if ztail:
            pltpu.sync_copy(rows_v.at[pl.ds(0, ztail)],
                            acc_sh.at[pl.ds(row0 + zrounds * 16, ztail)])

        plsc.subcore_barrier()

        ebase = wid * ept

        def _chunk(i, carry):
            base = ebase + i * CHUNK
            pltpu.sync_copy(gsrc.at[pl.ds(base, CHUNK)], gidx_v)
            pltpu.sync_copy(dst.at[pl.ds(base, CHUNK)], dst_v)
            pltpu.async_copy(hf.at[gidx_v], rows_v, sem).wait()
            pltpu.sync_copy(rows_v, acc_sh.at[dst_v], add=True)
            return carry
        lax.fori_loop(0, nchunks, _chunk, 0)

        plsc.subcore_barrier()

        def _wout(i, carry):
            r0 = row0 + i * 16
            pltpu.sync_copy(acc_sh.at[pl.ds(r0, 16)], zrows)
            pltpu.sync_copy(zrows, acc_out.at[c, pl.ds(r0, 16)])
            return carry
        lax.fori_loop(0, zrounds, _wout, 0)
        if ztail:
            r0 = row0 + zrounds * 16
            pltpu.sync_copy(acc_sh.at[pl.ds(r0, ztail)],
                            rows_v.at[pl.ds(0, ztail)])
            pltpu.sync_copy(rows_v.at[pl.ds(0, ztail)],
                            acc_out.at[c, pl.ds(r0, ztail)])

    return sweep


def _make_deg_sweep(np_, ept):
    nchunks = ept // CHUNK
    rows_pt = np_ // NS
    zrounds = rows_pt // 16
    ztail = rows_pt - zrounds * 16

    mesh = plsc.VectorSubcoreMesh(core_axis_name="c", subcore_axis_name="s")

    @functools.partial(
        pl.kernel,
        out_type=[
            jax.ShapeDtypeStruct((NC, np_, DEGW), jnp.float32),
        ],
        mesh=mesh,
        scratch_types=[
            pltpu.VMEM((CHUNK,), jnp.int32),
            pltpu.VMEM((CHUNK, DEGW), jnp.float32),
            pltpu.VMEM((16, DEGW), jnp.float32),
            pltpu.VMEM_SHARED((np_, DEGW), jnp.float32),
        ],
    )
    def dsweep(dst, deg_out, dst_v, ones_v, zd_v, deg_sh):
        c = lax.axis_index("c")
        s = lax.axis_index("s")
        wid = s * NC + c

        zf = jnp.zeros((LANES,), jnp.float32)
        ones = jnp.ones((LANES,), jnp.float32)
        for j in range(16):
            for k in range(DEGW // LANES):
                zd_v[j, pl.ds(k * LANES, LANES)] = zf
        for j in range(CHUNK):
            for k in range(DEGW // LANES):
                ones_v[j, pl.ds(k * LANES, LANES)] = ones

        row0 = s * rows_pt
        zrows = zd_v

        def _zacc(i, carry):
            pltpu.sync_copy(zrows, deg_sh.at[pl.ds(row0 + i * 16, 16)])
            return carry
        lax.fori_loop(0, zrounds, _zacc, 0)
        if ztail:
            pltpu.sync_copy(zd_v.at[pl.ds(0, ztail)],
                            deg_sh.at[pl.ds(row0 + zrounds * 16, ztail)])

        plsc.subcore_barrier()

        ebase = wid * ept

        def _chunk(i, carry):
            base = ebase + i * CHUNK
            pltpu.sync_copy(dst.at[pl.ds(base, CHUNK)], dst_v)
            pltpu.sync_copy(ones_v, deg_sh.at[dst_v], add=True)
            return carry
        lax.fori_loop(0, nchunks, _chunk, 0)

        plsc.subcore_barrier()

        def _wout(i, carry):
            r0 = row0 + i * 16
            pltpu.sync_copy(deg_sh.at[pl.ds(r0, 16)], zd_v)
            pltpu.sync_copy(zd_v, deg_out.at[c, pl.ds(r0, 16)])
            return carry
        lax.fori_loop(0, zrounds, _wout, 0)
        if ztail:
            r0 = row0 + zrounds * 16
            pltpu.sync_copy(deg_sh.at[pl.ds(r0, ztail)],
                            zd_v.at[pl.ds(0, ztail)])
            pltpu.sync_copy(zd_v.at[pl.ds(0, ztail)],
                            deg_out.at[c, pl.ds(r0, ztail)])

    return dsweep


def _finalize(acc, deg_p, x, lin_W, lin_b, gamma, beta, bm):
    n, d = x.shape

    def body(acc_ref, deg_ref, x_ref, w_ref, b_ref, g_ref, be_ref, o_ref):
        agg = acc_ref[0] + acc_ref[1]
        deg = jnp.maximum(deg_ref[0, :, 0:1] + deg_ref[1, :, 0:1], 1.0)
        h = lax.dot_general(x_ref[...], w_ref[...], (((1,), (1,)), ((), ())),
                            preferred_element_type=jnp.float32)
        h = h + b_ref[0] + agg / deg
        mean = jnp.mean(h, axis=1, keepdims=True)
        var = jnp.mean((h - mean) ** 2, axis=1, keepdims=True)
        o_ref[...] = (h - mean) * lax.rsqrt(var + 1e-5) * g_ref[0] + be_ref[0]

    return pl.pallas_call(
        body,
        grid=(n // bm,),
        in_specs=[
            pl.BlockSpec((2, bm, d), lambda i: (0, i, 0)),
            pl.BlockSpec((2, bm, DEGW), lambda i: (0, i, 0)),
            pl.BlockSpec((bm, d), lambda i: (i, 0)),
            pl.BlockSpec((d, d), lambda i: (0, 0)),
            pl.BlockSpec((1, d), lambda i: (0, 0)),
            pl.BlockSpec((1, d), lambda i: (0, 0)),
            pl.BlockSpec((1, d), lambda i: (0, 0)),
        ],
        out_specs=pl.BlockSpec((bm, d), lambda i: (i, 0)),
        out_shape=jax.ShapeDtypeStruct((n, d), jnp.float32),
    )(acc, deg_p, x, lin_W, lin_b.reshape(1, d), gamma.reshape(1, d),
      beta.reshape(1, d))


def kernel(x, edge_index, edge_type, relation_weights, lin_W, lin_b,
           gamma, beta):
    n, d = x.shape
    e = edge_index.shape[1]

    np_ = ((n + 8 + 127) // 128) * 128
    per_round = NC * NS * CHUNK
    ep = ((e + per_round - 1) // per_round) * per_round
    ept = ep // (NC * NS)

    gsrc = edge_type * n + edge_index[0]
    dst = edge_index[1]
    if ep != e:
        pad = ep - e
        gsrc = jnp.concatenate([gsrc, jnp.zeros((pad,), jnp.int32)])
        dst = jnp.concatenate([dst, jnp.full((pad,), n, jnp.int32)])

    h_tab = _make_tables(x, relation_weights, bn=2000)
    hf = h_tab.reshape(relation_weights.shape[0] * n, d)

    (acc,) = _make_edge_sweep(np_, d, ept)(hf, gsrc, dst)
    (degp,) = _make_deg_sweep(np_, ept)(dst)

    return _finalize(acc[:, :n, :], degp[:, :n, :], x, lin_W, lin_b,
                     gamma, beta, bm=2000)


# speedup vs baseline: 8.7425x; 1.3210x over previous
import functools

import jax
import jax.numpy as jnp
from jax import lax
from jax.experimental import pallas as pl
from jax.experimental.pallas import tpu as pltpu
from jax.experimental.pallas import tpu_sc as plsc

NC = 2
NS = 16
LANES = 16
DEGW = 128
CHUNK = 80


def _make_tables(x, relation_weights, bn):
    n, d = x.shape
    r = relation_weights.shape[0]

    def body(x_ref, w_ref, o_ref):
        o_ref[0] = jnp.dot(x_ref[...], w_ref[0],
                           preferred_element_type=jnp.float32)

    return pl.pallas_call(
        body,
        grid=(r, n // bn),
        in_specs=[
            pl.BlockSpec((bn, d), lambda i, j: (j, 0)),
            pl.BlockSpec((1, d, d), lambda i, j: (i, 0, 0)),
        ],
        out_specs=pl.BlockSpec((1, bn, d), lambda i, j: (i, j, 0)),
        out_shape=jax.ShapeDtypeStruct((r, n, d), jnp.float32),
    )(x, relation_weights)


def _make_edge_sweep(np_, d2, ept):
    nchunks = ept // CHUNK
    rows_pt = np_ // NS
    zrounds = rows_pt // 16
    ztail = rows_pt - zrounds * 16

    mesh = plsc.VectorSubcoreMesh(core_axis_name="c", subcore_axis_name="s")

    assert nchunks % 2 == 1, "pipeline below assumes an odd chunk count"
    npairs = nchunks // 2

    @functools.partial(
        pl.kernel,
        out_type=[
            jax.ShapeDtypeStruct((NC, np_, d2), jnp.float32),
        ],
        mesh=mesh,
        scratch_types=[
            pltpu.VMEM((2, CHUNK), jnp.int32),
            pltpu.VMEM((2, CHUNK), jnp.int32),
            pltpu.VMEM((2, CHUNK, d2), jnp.float32),
            pltpu.VMEM_SHARED((np_, d2), jnp.float32),
            pltpu.SemaphoreType.DMA((2,)),
        ],
    )
    def sweep(hf, gsrc, dst, acc_out, gidx_v, dst_v, rows_v, acc_sh, sem):
        c = lax.axis_index("c")
        s = lax.axis_index("s")
        wid = s * NC + c

        zf = jnp.zeros((LANES,), jnp.float32)

        for j in range(16):
            for k in range(d2 // LANES):
                rows_v[0, j, pl.ds(k * LANES, LANES)] = zf

        row0 = s * rows_pt
        zrows = rows_v.at[0, pl.ds(0, 16)]

        def _zacc(i, carry):
            pltpu.sync_copy(zrows, acc_sh.at[pl.ds(row0 + i * 16, 16)])
            return carry
        lax.fori_loop(0, zrounds, _zacc, 0)
        if ztail:
            pltpu.sync_copy(rows_v.at[0, pl.ds(0, ztail)],
                            acc_sh.at[pl.ds(row0 + zrounds * 16, ztail)])

        plsc.subcore_barrier()

        ebase = wid * ept
        r0b = [rows_v.at[0], rows_v.at[1]]
        g0b = [gidx_v.at[0], gidx_v.at[1]]
        d0b = [dst_v.at[0], dst_v.at[1]]

        def _stage(b, i):
            base = ebase + i * CHUNK
            pltpu.sync_copy(gsrc.at[pl.ds(base, CHUNK)], g0b[b])
            pltpu.sync_copy(dst.at[pl.ds(base, CHUNK)], d0b[b])

        def _gather_start(b):
            pltpu.async_copy(hf.at[g0b[b]], r0b[b], sem.at[b])

        def _gather_wait(b):
            pltpu.make_async_copy(hf.at[g0b[b]], r0b[b], sem.at[b]).wait()

        def _scatter(b):
            pltpu.sync_copy(r0b[b], acc_sh.at[d0b[b]], add=True)

        _stage(0, 0)
        _stage(1, 1)
        _gather_start(0)

        def _pair(j, carry):
            i = j * 2
            _gather_wait(0)
            _gather_start(1)
            _scatter(0)
            _stage(0, i + 2)
            _gather_wait(1)
            _gather_start(0)
            _scatter(1)
            _stage(1, i + 3)
            return carry
        lax.fori_loop(0, npairs, _pair, 0)
        _gather_wait(0)
        _scatter(0)

        plsc.subcore_barrier()

        def _wout(i, carry):
            r0 = row0 + i * 16
            pltpu.sync_copy(acc_sh.at[pl.ds(r0, 16)], zrows)
            pltpu.sync_copy(zrows, acc_out.at[c, pl.ds(r0, 16)])
            return carry
        lax.fori_loop(0, zrounds, _wout, 0)
        if ztail:
            r0 = row0 + zrounds * 16
            pltpu.sync_copy(acc_sh.at[pl.ds(r0, ztail)],
                            rows_v.at[0, pl.ds(0, ztail)])
            pltpu.sync_copy(rows_v.at[0, pl.ds(0, ztail)],
                            acc_out.at[c, pl.ds(r0, ztail)])

    return sweep


def _make_deg_sweep(np_, ept):
    nchunks = ept // CHUNK
    rows_pt = np_ // NS
    zrounds = rows_pt // 16
    ztail = rows_pt - zrounds * 16

    mesh = plsc.VectorSubcoreMesh(core_axis_name="c", subcore_axis_name="s")

    @functools.partial(
        pl.kernel,
        out_type=[
            jax.ShapeDtypeStruct((NC, np_, DEGW), jnp.float32),
        ],
        mesh=mesh,
        scratch_types=[
            pltpu.VMEM((CHUNK,), jnp.int32),
            pltpu.VMEM((CHUNK, DEGW), jnp.float32),
            pltpu.VMEM((16, DEGW), jnp.float32),
            pltpu.VMEM_SHARED((np_, DEGW), jnp.float32),
        ],
    )
    def dsweep(dst, deg_out, dst_v, ones_v, zd_v, deg_sh):
        c = lax.axis_index("c")
        s = lax.axis_index("s")
        wid = s * NC + c

        zf = jnp.zeros((LANES,), jnp.float32)
        ones = jnp.ones((LANES,), jnp.float32)
        for j in range(16):
            for k in range(DEGW // LANES):
                zd_v[j, pl.ds(k * LANES, LANES)] = zf
        for j in range(CHUNK):
            for k in range(DEGW // LANES):
                ones_v[j, pl.ds(k * LANES, LANES)] = ones

        row0 = s * rows_pt
        zrows = zd_v

        def _zacc(i, carry):
            pltpu.sync_copy(zrows, deg_sh.at[pl.ds(row0 + i * 16, 16)])
            return carry
        lax.fori_loop(0, zrounds, _zacc, 0)
        if ztail:
            pltpu.sync_copy(zd_v.at[pl.ds(0, ztail)],
                            deg_sh.at[pl.ds(row0 + zrounds * 16, ztail)])

        plsc.subcore_barrier()

        ebase = wid * ept

        def _chunk(i, carry):
            base = ebase + i * CHUNK
            pltpu.sync_copy(dst.at[pl.ds(base, CHUNK)], dst_v)
            pltpu.sync_copy(ones_v, deg_sh.at[dst_v], add=True)
            return carry
        lax.fori_loop(0, nchunks, _chunk, 0)

        plsc.subcore_barrier()

        def _wout(i, carry):
            r0 = row0 + i * 16
            pltpu.sync_copy(deg_sh.at[pl.ds(r0, 16)], zd_v)
            pltpu.sync_copy(zd_v, deg_out.at[c, pl.ds(r0, 16)])
            return carry
        lax.fori_loop(0, zrounds, _wout, 0)
        if ztail:
            r0 = row0 + zrounds * 16
            pltpu.sync_copy(deg_sh.at[pl.ds(r0, ztail)],
                            zd_v.at[pl.ds(0, ztail)])
            pltpu.sync_copy(zd_v.at[pl.ds(0, ztail)],
                            deg_out.at[c, pl.ds(r0, ztail)])

    return dsweep


def _finalize(acc, deg_p, x, lin_W, lin_b, gamma, beta, bm):
    n, d = x.shape

    def body(acc_ref, deg_ref, x_ref, w_ref, b_ref, g_ref, be_ref, o_ref):
        agg = acc_ref[0] + acc_ref[1]
        deg = jnp.maximum(deg_ref[0, :, 0:1] + deg_ref[1, :, 0:1], 1.0)
        h = lax.dot_general(x_ref[...], w_ref[...], (((1,), (1,)), ((), ())),
                            preferred_element_type=jnp.float32)
        h = h + b_ref[0] + agg / deg
        mean = jnp.mean(h, axis=1, keepdims=True)
        var = jnp.mean((h - mean) ** 2, axis=1, keepdims=True)
        o_ref[...] = (h - mean) * lax.rsqrt(var + 1e-5) * g_ref[0] + be_ref[0]

    return pl.pallas_call(
        body,
        grid=(n // bm,),
        in_specs=[
            pl.BlockSpec((2, bm, d), lambda i: (0, i, 0)),
            pl.BlockSpec((2, bm, DEGW), lambda i: (0, i, 0)),
            pl.BlockSpec((bm, d), lambda i: (i, 0)),
            pl.BlockSpec((d, d), lambda i: (0, 0)),
            pl.BlockSpec((1, d), lambda i: (0, 0)),
            pl.BlockSpec((1, d), lambda i: (0, 0)),
            pl.BlockSpec((1, d), lambda i: (0, 0)),
        ],
        out_specs=pl.BlockSpec((bm, d), lambda i: (i, 0)),
        out_shape=jax.ShapeDtypeStruct((n, d), jnp.float32),
    )(acc, deg_p, x, lin_W, lin_b.reshape(1, d), gamma.reshape(1, d),
      beta.reshape(1, d))


def kernel(x, edge_index, edge_type, relation_weights, lin_W, lin_b,
           gamma, beta):
    n, d = x.shape
    e = edge_index.shape[1]

    np_ = ((n + 8 + 127) // 128) * 128
    per_round = NC * NS * CHUNK
    ep = ((e + per_round - 1) // per_round) * per_round
    ept = ep // (NC * NS)

    gsrc = edge_type * n + edge_index[0]
    dst = edge_index[1]
    pad = ep - e + CHUNK
    gsrc = jnp.concatenate([gsrc, jnp.zeros((pad,), jnp.int32)])
    dst = jnp.concatenate([dst, jnp.full((pad,), n, jnp.int32)])

    h_tab = _make_tables(x, relation_weights, bn=2000)
    hf = h_tab.reshape(relation_weights.shape[0] * n, d)

    (acc,) = _make_edge_sweep(np_, d, ept)(hf, gsrc, dst)
    (degp,) = _make_deg_sweep(np_, ept)(dst)

    return _finalize(acc[:, :n, :], degp[:, :n, :], x, lin_W, lin_b,
                     gamma, beta, bm=2000)


# speedup vs baseline: 8.9565x; 1.0245x over previous
import functools

import jax
import jax.numpy as jnp
from jax import lax
from jax.experimental import pallas as pl
from jax.experimental.pallas import tpu as pltpu
from jax.experimental.pallas import tpu_sc as plsc

NC = 2
NS = 16
LANES = 16
DEGW = 128
CHUNK = 80


def _make_tables(x, relation_weights, bn):
    n, d = x.shape
    r = relation_weights.shape[0]

    def body(x_ref, w_ref, o_ref):
        o_ref[0] = jnp.dot(x_ref[...], w_ref[0],
                           preferred_element_type=jnp.float32)

    return pl.pallas_call(
        body,
        grid=(r, n // bn),
        in_specs=[
            pl.BlockSpec((bn, d), lambda i, j: (j, 0)),
            pl.BlockSpec((1, d, d), lambda i, j: (i, 0, 0)),
        ],
        out_specs=pl.BlockSpec((1, bn, d), lambda i, j: (i, j, 0)),
        out_shape=jax.ShapeDtypeStruct((r, n, d), jnp.float32),
    )(x, relation_weights)


def _make_edge_sweep(np_, d2, ept):
    nchunks = ept // CHUNK
    rows_pt = np_ // NS
    zrounds = rows_pt // 16
    ztail = rows_pt - zrounds * 16

    mesh = plsc.VectorSubcoreMesh(core_axis_name="c", subcore_axis_name="s")

    assert nchunks % 2 == 1, "pipeline below assumes an odd chunk count"
    npairs = nchunks // 2

    @functools.partial(
        pl.kernel,
        out_type=[
            jax.ShapeDtypeStruct((NC, np_, d2), jnp.float32),
        ],
        mesh=mesh,
        scratch_types=[
            pltpu.VMEM((2, CHUNK), jnp.int32),
            pltpu.VMEM((2, CHUNK), jnp.int32),
            pltpu.VMEM((2, CHUNK, d2), jnp.float32),
            pltpu.VMEM_SHARED((np_, d2), jnp.float32),
            pltpu.SemaphoreType.DMA((2,)),
        ],
    )
    def sweep(hf, gsrc, dst, acc_out, gidx_v, dst_v, rows_v, acc_sh, sem):
        c = lax.axis_index("c")
        s = lax.axis_index("s")
        wid = s * NC + c

        zf = jnp.zeros((LANES,), jnp.float32)

        for j in range(16):
            for k in range(d2 // LANES):
                rows_v[0, j, pl.ds(k * LANES, LANES)] = zf

        row0 = s * rows_pt
        zrows = rows_v.at[0, pl.ds(0, 16)]

        def _zacc(i, carry):
            pltpu.sync_copy(zrows, acc_sh.at[pl.ds(row0 + i * 16, 16)])
            return carry
        lax.fori_loop(0, zrounds, _zacc, 0)
        if ztail:
            pltpu.sync_copy(rows_v.at[0, pl.ds(0, ztail)],
                            acc_sh.at[pl.ds(row0 + zrounds * 16, ztail)])

        plsc.subcore_barrier()

        ebase = wid * ept
        r0b = [rows_v.at[0], rows_v.at[1]]
        g0b = [gidx_v.at[0], gidx_v.at[1]]
        d0b = [dst_v.at[0], dst_v.at[1]]

        def _stage(b, i):
            base = ebase + i * CHUNK
            pltpu.sync_copy(gsrc.at[pl.ds(base, CHUNK)], g0b[b])
            pltpu.sync_copy(dst.at[pl.ds(base, CHUNK)], d0b[b])

        def _gather_start(b):
            pltpu.async_copy(hf.at[g0b[b]], r0b[b], sem.at[b])

        def _gather_wait(b):
            pltpu.make_async_copy(hf.at[g0b[b]], r0b[b], sem.at[b]).wait()

        def _scatter(b):
            pltpu.sync_copy(r0b[b], acc_sh.at[d0b[b]], add=True)

        _stage(0, 0)
        _stage(1, 1)
        _gather_start(0)

        def _pair(j, carry):
            i = j * 2
            _gather_wait(0)
            _gather_start(1)
            _scatter(0)
            _stage(0, i + 2)
            _gather_wait(1)
            _gather_start(0)
            _scatter(1)
            _stage(1, i + 3)
            return carry
        lax.fori_loop(0, npairs, _pair, 0)
        _gather_wait(0)
        _scatter(0)

        plsc.subcore_barrier()

        def _wout(i, carry):
            r0 = row0 + i * 16
            pltpu.sync_copy(acc_sh.at[pl.ds(r0, 16)], zrows)
            pltpu.sync_copy(zrows, acc_out.at[c, pl.ds(r0, 16)])
            return carry
        lax.fori_loop(0, zrounds, _wout, 0)
        if ztail:
            r0 = row0 + zrounds * 16
            pltpu.sync_copy(acc_sh.at[pl.ds(r0, ztail)],
                            rows_v.at[0, pl.ds(0, ztail)])
            pltpu.sync_copy(rows_v.at[0, pl.ds(0, ztail)],
                            acc_out.at[c, pl.ds(r0, ztail)])

    return sweep


def _make_deg_sweep(np_, ept):
    nchunks = ept // CHUNK
    rows_pt = np_ // NS
    zrounds = rows_pt // 16
    ztail = rows_pt - zrounds * 16

    mesh = plsc.VectorSubcoreMesh(core_axis_name="c", subcore_axis_name="s")

    @functools.partial(
        pl.kernel,
        out_type=[
            jax.ShapeDtypeStruct((NC, np_, DEGW), jnp.float32),
        ],
        mesh=mesh,
        scratch_types=[
            pltpu.VMEM((CHUNK,), jnp.int32),
            pltpu.VMEM((CHUNK, DEGW), jnp.float32),
            pltpu.VMEM((16, DEGW), jnp.float32),
            pltpu.VMEM_SHARED((np_, DEGW), jnp.float32),
        ],
    )
    def dsweep(dst, deg_out, dst_v, ones_v, zd_v, deg_sh):
        c = lax.axis_index("c")
        s = lax.axis_index("s")
        wid = s * NC + c

        zf = jnp.zeros((LANES,), jnp.float32)
        ones = jnp.ones((LANES,), jnp.float32)
        for j in range(16):
            for k in range(DEGW // LANES):
                zd_v[j, pl.ds(k * LANES, LANES)] = zf
        for j in range(CHUNK):
            for k in range(DEGW // LANES):
                ones_v[j, pl.ds(k * LANES, LANES)] = ones

        row0 = s * rows_pt
        zrows = zd_v

        def _zacc(i, carry):
            pltpu.sync_copy(zrows, deg_sh.at[pl.ds(row0 + i * 16, 16)])
            return carry
        lax.fori_loop(0, zrounds, _zacc, 0)
        if ztail:
            pltpu.sync_copy(zd_v.at[pl.ds(0, ztail)],
                            deg_sh.at[pl.ds(row0 + zrounds * 16, ztail)])

        plsc.subcore_barrier()

        ebase = wid * ept

        def _chunk(i, carry):
            base = ebase + i * CHUNK
            pltpu.sync_copy(dst.at[pl.ds(base, CHUNK)], dst_v)
            pltpu.sync_copy(ones_v, deg_sh.at[dst_v], add=True)
            return carry
        lax.fori_loop(0, nchunks, _chunk, 0)

        plsc.subcore_barrier()

        def _wout(i, carry):
            r0 = row0 + i * 16
            pltpu.sync_copy(deg_sh.at[pl.ds(r0, 16)], zd_v)
            pltpu.sync_copy(zd_v, deg_out.at[c, pl.ds(r0, 16)])
            return carry
        lax.fori_loop(0, zrounds, _wout, 0)
        if ztail:
            r0 = row0 + zrounds * 16
            pltpu.sync_copy(deg_sh.at[pl.ds(r0, ztail)],
                            zd_v.at[pl.ds(0, ztail)])
            pltpu.sync_copy(zd_v.at[pl.ds(0, ztail)],
                            deg_out.at[c, pl.ds(r0, ztail)])

    return dsweep


def _finalize(acc, deg_p, x, lin_W, lin_b, gamma, beta, bm):
    n, d = x.shape

    def body(acc_ref, deg_ref, x_ref, w_ref, b_ref, g_ref, be_ref, o_ref):
        agg = acc_ref[0] + acc_ref[1]
        deg = jnp.maximum(deg_ref[0, :, 0:1] + deg_ref[1, :, 0:1], 1.0)
        h = lax.dot_general(x_ref[...], w_ref[...], (((1,), (1,)), ((), ())),
                            preferred_element_type=jnp.float32)
        h = h + b_ref[0] + agg / deg
        mean = jnp.mean(h, axis=1, keepdims=True)
        var = jnp.mean((h - mean) ** 2, axis=1, keepdims=True)
        o_ref[...] = (h - mean) * lax.rsqrt(var + 1e-5) * g_ref[0] + be_ref[0]

    return pl.pallas_call(
        body,
        grid=(n // bm,),
        in_specs=[
            pl.BlockSpec((2, bm, d), lambda i: (0, i, 0)),
            pl.BlockSpec((2, bm, DEGW), lambda i: (0, i, 0)),
            pl.BlockSpec((bm, d), lambda i: (i, 0)),
            pl.BlockSpec((d, d), lambda i: (0, 0)),
            pl.BlockSpec((1, d), lambda i: (0, 0)),
            pl.BlockSpec((1, d), lambda i: (0, 0)),
            pl.BlockSpec((1, d), lambda i: (0, 0)),
        ],
        out_specs=pl.BlockSpec((bm, d), lambda i: (i, 0)),
        out_shape=jax.ShapeDtypeStruct((n, d), jnp.float32),
    )(acc, deg_p, x, lin_W, lin_b.reshape(1, d), gamma.reshape(1, d),
      beta.reshape(1, d))


def kernel(x, edge_index, edge_type, relation_weights, lin_W, lin_b,
           gamma, beta):
    n, d = x.shape
    e = edge_index.shape[1]

    np_ = ((n + 8 + 127) // 128) * 128
    per_round = NC * NS * CHUNK
    ep = ((e + per_round - 1) // per_round) * per_round
    ept = ep // (NC * NS)

    gsrc = edge_type * n + edge_index[0]
    dst = edge_index[1]
    pad = ep - e + CHUNK
    gsrc = jnp.concatenate([gsrc, jnp.zeros((pad,), jnp.int32)])
    dst = jnp.concatenate([dst, jnp.full((pad,), n, jnp.int32)])

    h_tab = _make_tables(x, relation_weights, bn=2000)
    hf = h_tab.reshape(relation_weights.shape[0] * n, d)

    (acc,) = _make_edge_sweep(np_, d, ept)(hf, gsrc, dst)
    (degp,) = _make_deg_sweep(np_, ept)(dst)

    return _finalize(acc, degp, x, lin_W, lin_b, gamma, beta, bm=2000)


# speedup vs baseline: 11.2850x; 1.2600x over previous
import functools

import jax
import jax.numpy as jnp
from jax import lax
from jax.experimental import pallas as pl
from jax.experimental.pallas import tpu as pltpu
from jax.experimental.pallas import tpu_sc as plsc

NC = 2
NS = 16
LANES = 16
DEGW = 128
CHUNK = 80


def _make_tables(x, relation_weights, bn):
    n, d = x.shape
    r = relation_weights.shape[0]

    def body(x_ref, w_ref, o_ref):
        o_ref[0] = jnp.dot(x_ref[...], w_ref[0],
                           preferred_element_type=jnp.float32)

    return pl.pallas_call(
        body,
        grid=(r, n // bn),
        in_specs=[
            pl.BlockSpec((bn, d), lambda i, j: (j, 0)),
            pl.BlockSpec((1, d, d), lambda i, j: (i, 0, 0)),
        ],
        out_specs=pl.BlockSpec((1, bn, d), lambda i, j: (i, j, 0)),
        out_shape=jax.ShapeDtypeStruct((r, n, d), jnp.float32),
    )(x, relation_weights)


def _make_edge_sweep(np_, d2, ept):
    nchunks = ept // CHUNK
    rows_pt = np_ // NS
    zrounds = rows_pt // 16
    ztail = rows_pt - zrounds * 16

    mesh = plsc.VectorSubcoreMesh(core_axis_name="c", subcore_axis_name="s")

    assert nchunks % 2 == 1, "pipeline below assumes an odd chunk count"
    npairs = nchunks // 2

    @functools.partial(
        pl.kernel,
        out_type=[
            jax.ShapeDtypeStruct((NC, np_, d2), jnp.float32),
        ],
        mesh=mesh,
        scratch_types=[
            pltpu.VMEM((2, CHUNK), jnp.int32),
            pltpu.VMEM((2, CHUNK), jnp.int32),
            pltpu.VMEM((2, CHUNK, d2), jnp.float32),
            pltpu.VMEM_SHARED((np_, d2), jnp.float32),
            pltpu.SemaphoreType.DMA((6,)),
        ],
    )
    def sweep(hf, gsrc, dst, acc_out, gidx_v, dst_v, rows_v, acc_sh, sem):
        c = lax.axis_index("c")
        s = lax.axis_index("s")
        wid = s * NC + c

        zf = jnp.zeros((LANES,), jnp.float32)

        for j in range(16):
            for k in range(d2 // LANES):
                rows_v[0, j, pl.ds(k * LANES, LANES)] = zf

        row0 = s * rows_pt
        zrows = rows_v.at[0, pl.ds(0, 16)]

        def _zacc(i, carry):
            pltpu.sync_copy(zrows, acc_sh.at[pl.ds(row0 + i * 16, 16)])
            return carry
        lax.fori_loop(0, zrounds, _zacc, 0)
        if ztail:
            pltpu.sync_copy(rows_v.at[0, pl.ds(0, ztail)],
                            acc_sh.at[pl.ds(row0 + zrounds * 16, ztail)])

        plsc.subcore_barrier()

        ebase = wid * ept
        r0b = [rows_v.at[0], rows_v.at[1]]
        g0b = [gidx_v.at[0], gidx_v.at[1]]
        d0b = [dst_v.at[0], dst_v.at[1]]

        def _stage_start(b, i):
            base = ebase + i * CHUNK
            pltpu.async_copy(gsrc.at[pl.ds(base, CHUNK)], g0b[b],
                             sem.at[2 + b])
            pltpu.async_copy(dst.at[pl.ds(base, CHUNK)], d0b[b],
                             sem.at[4 + b])

        def _stage_wait(b):
            base = ebase
            pltpu.make_async_copy(gsrc.at[pl.ds(base, CHUNK)], g0b[b],
                                  sem.at[2 + b]).wait()
            pltpu.make_async_copy(dst.at[pl.ds(base, CHUNK)], d0b[b],
                                  sem.at[4 + b]).wait()

        def _gather_start(b):
            pltpu.async_copy(hf.at[g0b[b]], r0b[b], sem.at[b])

        def _gather_wait(b):
            pltpu.make_async_copy(hf.at[g0b[b]], r0b[b], sem.at[b]).wait()

        def _scatter(b):
            pltpu.sync_copy(r0b[b], acc_sh.at[d0b[b]], add=True)

        _stage_start(0, 0)
        _stage_wait(0)
        _stage_start(1, 1)
        _gather_start(0)

        def _pair(j, carry):
            i = j * 2
            _gather_wait(0)
            _stage_wait(1)
            _gather_start(1)
            _scatter(0)
            _stage_start(0, i + 2)
            _gather_wait(1)
            _stage_wait(0)
            _gather_start(0)
            _scatter(1)
            _stage_start(1, i + 3)
            return carry
        lax.fori_loop(0, npairs, _pair, 0)
        _gather_wait(0)
        _scatter(0)
        _stage_wait(1)

        plsc.subcore_barrier()

        def _wout(i, carry):
            r0 = row0 + i * 16
            pltpu.sync_copy(acc_sh.at[pl.ds(r0, 16)], zrows)
            pltpu.sync_copy(zrows, acc_out.at[c, pl.ds(r0, 16)])
            return carry
        lax.fori_loop(0, zrounds, _wout, 0)
        if ztail:
            r0 = row0 + zrounds * 16
            pltpu.sync_copy(acc_sh.at[pl.ds(r0, ztail)],
                            rows_v.at[0, pl.ds(0, ztail)])
            pltpu.sync_copy(rows_v.at[0, pl.ds(0, ztail)],
                            acc_out.at[c, pl.ds(r0, ztail)])

    return sweep


def _make_deg_sweep(np_, ept):
    nchunks = ept // CHUNK
    rows_pt = np_ // NS
    zrounds = rows_pt // 16
    ztail = rows_pt - zrounds * 16

    mesh = plsc.VectorSubcoreMesh(core_axis_name="c", subcore_axis_name="s")

    @functools.partial(
        pl.kernel,
        out_type=[
            jax.ShapeDtypeStruct((NC, np_, DEGW), jnp.float32),
        ],
        mesh=mesh,
        scratch_types=[
            pltpu.VMEM((2, CHUNK), jnp.int32),
            pltpu.VMEM((CHUNK, DEGW), jnp.float32),
            pltpu.VMEM((16, DEGW), jnp.float32),
            pltpu.VMEM_SHARED((np_, DEGW), jnp.float32),
            pltpu.SemaphoreType.DMA((2,)),
        ],
    )
    def dsweep(dst, deg_out, dst_v, ones_v, zd_v, deg_sh, sem):
        c = lax.axis_index("c")
        s = lax.axis_index("s")
        wid = s * NC + c

        zf = jnp.zeros((LANES,), jnp.float32)
        ones = jnp.ones((LANES,), jnp.float32)
        for j in range(16):
            for k in range(DEGW // LANES):
                zd_v[j, pl.ds(k * LANES, LANES)] = zf
        for j in range(CHUNK):
            for k in range(DEGW // LANES):
                ones_v[j, pl.ds(k * LANES, LANES)] = ones

        row0 = s * rows_pt
        zrows = zd_v

        def _zacc(i, carry):
            pltpu.sync_copy(zrows, deg_sh.at[pl.ds(row0 + i * 16, 16)])
            return carry
        lax.fori_loop(0, zrounds, _zacc, 0)
        if ztail:
            pltpu.sync_copy(zd_v.at[pl.ds(0, ztail)],
                            deg_sh.at[pl.ds(row0 + zrounds * 16, ztail)])

        plsc.subcore_barrier()

        ebase = wid * ept
        d0b = [dst_v.at[0], dst_v.at[1]]

        def _stage_start(b, i):
            pltpu.async_copy(dst.at[pl.ds(ebase + i * CHUNK, CHUNK)],
                             d0b[b], sem.at[b])

        def _stage_wait(b):
            pltpu.make_async_copy(dst.at[pl.ds(ebase, CHUNK)], d0b[b],
                                  sem.at[b]).wait()

        def _scatter(b):
            pltpu.sync_copy(ones_v, deg_sh.at[d0b[b]], add=True)

        _stage_start(0, 0)

        def _pair(j, carry):
            i = j * 2
            _stage_wait(0)
            _stage_start(1, i + 1)
            _scatter(0)
            _stage_wait(1)
            _stage_start(0, i + 2)
            _scatter(1)
            return carry
        lax.fori_loop(0, nchunks // 2, _pair, 0)
        _stage_wait(0)
        _scatter(0)

        plsc.subcore_barrier()

        def _wout(i, carry):
            r0 = row0 + i * 16
            pltpu.sync_copy(deg_sh.at[pl.ds(r0, 16)], zd_v)
            pltpu.sync_copy(zd_v, deg_out.at[c, pl.ds(r0, 16)])
            return carry
        lax.fori_loop(0, zrounds, _wout, 0)
        if ztail:
            r0 = row0 + zrounds * 16
            pltpu.sync_copy(deg_sh.at[pl.ds(r0, ztail)],
                            zd_v.at[pl.ds(0, ztail)])
            pltpu.sync_copy(zd_v.at[pl.ds(0, ztail)],
                            deg_out.at[c, pl.ds(r0, ztail)])

    return dsweep


def _finalize(acc, deg_p, x, lin_W, lin_b, gamma, beta, bm):
    n, d = x.shape

    def body(acc_ref, deg_ref, x_ref, w_ref, b_ref, g_ref, be_ref, o_ref):
        agg = acc_ref[0] + acc_ref[1]
        deg = jnp.maximum(deg_ref[0, :, 0:1] + deg_ref[1, :, 0:1], 1.0)
        h = lax.dot_general(x_ref[...], w_ref[...], (((1,), (1,)), ((), ())),
                            preferred_element_type=jnp.float32)
        h = h + b_ref[0] + agg / deg
        mean = jnp.mean(h, axis=1, keepdims=True)
        var = jnp.mean((h - mean) ** 2, axis=1, keepdims=True)
        o_ref[...] = (h - mean) * lax.rsqrt(var + 1e-5) * g_ref[0] + be_ref[0]

    return pl.pallas_call(
        body,
        grid=(n // bm,),
        in_specs=[
            pl.BlockSpec((2, bm, d), lambda i: (0, i, 0)),
            pl.BlockSpec((2, bm, DEGW), lambda i: (0, i, 0)),
            pl.BlockSpec((bm, d), lambda i: (i, 0)),
            pl.BlockSpec((d, d), lambda i: (0, 0)),
            pl.BlockSpec((1, d), lambda i: (0, 0)),
            pl.BlockSpec((1, d), lambda i: (0, 0)),
            pl.BlockSpec((1, d), lambda i: (0, 0)),
        ],
        out_specs=pl.BlockSpec((bm, d), lambda i: (i, 0)),
        out_shape=jax.ShapeDtypeStruct((n, d), jnp.float32),
    )(acc, deg_p, x, lin_W, lin_b.reshape(1, d), gamma.reshape(1, d),
      beta.reshape(1, d))


def kernel(x, edge_index, edge_type, relation_weights, lin_W, lin_b,
           gamma, beta):
    n, d = x.shape
    e = edge_index.shape[1]

    np_ = ((n + 8 + 127) // 128) * 128
    per_round = NC * NS * CHUNK
    ep = ((e + per_round - 1) // per_round) * per_round
    ept = ep // (NC * NS)

    gsrc = edge_type * n + edge_index[0]
    dst = edge_index[1]
    pad = ep - e + CHUNK
    gsrc = jnp.concatenate([gsrc, jnp.zeros((pad,), jnp.int32)])
    dst = jnp.concatenate([dst, jnp.full((pad,), n, jnp.int32)])

    h_tab = _make_tables(x, relation_weights, bn=2000)
    hf = h_tab.reshape(relation_weights.shape[0] * n, d)

    (acc,) = _make_edge_sweep(np_, d, ept)(hf, gsrc, dst)
    (degp,) = _make_deg_sweep(np_, ept)(dst)

    return _finalize(acc, degp, x, lin_W, lin_b, gamma, beta, bm=2000)
